# baseline (speedup 1.0000x reference)
import functools

import jax
import jax.numpy as jnp
from jax.experimental import pallas as pl
from jax.experimental.pallas import tpu as pltpu


def _router_kernel(x_ref, w1_ref, b1_ref, w2_ref, b2_ref,
                   logits_ref, gate_ref, *, ch):
    hidden = w1_ref.shape[1]
    xb = x_ref[...]
    part = None
    for c in range(hidden // ch):
        cols = pl.ds(c * ch, ch)
        h = jnp.dot(xb, w1_ref[:, cols], preferred_element_type=jnp.float32)
        h = h + b1_ref[:, cols]
        h = h * jax.nn.sigmoid(h)
        p = jnp.dot(h.astype(jnp.bfloat16), w2_ref[cols, :],
                    preferred_element_type=jnp.float32)
        part = p if part is None else part + p

    logits = part + b2_ref[...]
    logits_ref[...] = logits
    m = jnp.max(logits, axis=-1, keepdims=True)
    e = jnp.exp(logits - m)
    gate_ref[...] = e / jnp.sum(e, axis=-1, keepdims=True)


@functools.partial(jax.jit, static_argnames=("tm", "ch"))
def _router(flow_input, W1, b1, W2, b2, tm=512, ch=1024):
    tokens, d_model = flow_input.shape
    hidden, num_experts = W2.shape
    tm = min(tm, tokens)
    ch = min(ch, hidden)
    ni = tokens // tm

    flow_input = flow_input.astype(jnp.bfloat16)
    W1 = W1.astype(jnp.bfloat16)
    W2 = W2.astype(jnp.bfloat16)
    b1_2d = b1.reshape(1, hidden)
    b2_2d = b2.reshape(1, num_experts)

    out_shapes = (
        jax.ShapeDtypeStruct((tokens, num_experts), jnp.float32),
        jax.ShapeDtypeStruct((tokens, num_experts), jnp.float32),
    )

    kernel_fn = functools.partial(_router_kernel, ch=ch)

    return pl.pallas_call(
        kernel_fn,
        grid=(ni,),
        in_specs=[
            pl.BlockSpec((tm, d_model), lambda i: (i, 0)),
            pl.BlockSpec((d_model, hidden), lambda i: (0, 0)),
            pl.BlockSpec((1, hidden), lambda i: (0, 0)),
            pl.BlockSpec((hidden, num_experts), lambda i: (0, 0)),
            pl.BlockSpec((1, num_experts), lambda i: (0, 0)),
        ],
        out_specs=[
            pl.BlockSpec((tm, num_experts), lambda i: (i, 0)),
            pl.BlockSpec((tm, num_experts), lambda i: (i, 0)),
        ],
        out_shape=out_shapes,
        compiler_params=pltpu.CompilerParams(
            dimension_semantics=("parallel",),
        ),
    )(flow_input, W1, b1_2d, W2, b2_2d)


def kernel(flow_input, W1, b1, W2, b2):
    return _router(flow_input, W1, b1, W2, b2)


# speedup vs baseline: 1.1764x; 1.1764x over previous
import functools

import jax
import jax.numpy as jnp
from jax.experimental import pallas as pl
from jax.experimental.pallas import tpu as pltpu


def _router_kernel(x_ref, w1_ref, b1_ref, w2_ref, b2_ref,
                   logits_ref, gate_ref, acc_ref):
    j = pl.program_id(1)
    nj = pl.num_programs(1)

    kd = x_ref.shape[1] // 2
    xb = x_ref[...].astype(jnp.bfloat16)
    h = (jnp.dot(xb[:, :kd], w1_ref[:kd, :],
                 preferred_element_type=jnp.float32) +
         jnp.dot(xb[:, kd:], w1_ref[kd:, :],
                 preferred_element_type=jnp.float32))
    h = h + b1_ref[...]
    h = h * jax.nn.sigmoid(h)
    part = jnp.dot(h.astype(jnp.bfloat16), w2_ref[...],
                   preferred_element_type=jnp.float32)

    @pl.when(j == 0)
    def _init():
        acc_ref[...] = part

    @pl.when(j != 0)
    def _accum():
        acc_ref[...] += part

    @pl.when(j == nj - 1)
    def _finish():
        logits = acc_ref[...] + b2_ref[...]
        logits_ref[...] = logits
        m = jnp.max(logits, axis=-1, keepdims=True)
        e = jnp.exp(logits - m)
        gate_ref[...] = e / jnp.sum(e, axis=-1, keepdims=True)


@functools.partial(jax.jit, static_argnames=("tm", "th"))
def _router(flow_input, W1, b1, W2, b2, tm=512, th=2048):
    tokens, d_model = flow_input.shape
    hidden, num_experts = W2.shape
    tm = min(tm, tokens)
    th = min(th, hidden)
    ni = tokens // tm
    nj = hidden // th

    W1 = W1.astype(jnp.bfloat16)
    W2 = W2.astype(jnp.bfloat16)
    b1_2d = b1.reshape(1, hidden)
    b2_2d = b2.reshape(1, num_experts)

    out_shapes = (
        jax.ShapeDtypeStruct((tokens, num_experts), jnp.float32),
        jax.ShapeDtypeStruct((tokens, num_experts), jnp.float32),
    )

    grid_spec = pltpu.PrefetchScalarGridSpec(
        num_scalar_prefetch=0,
        grid=(ni, nj),
        in_specs=[
            pl.BlockSpec((tm, d_model), lambda i, j: (i, 0)),
            pl.BlockSpec((d_model, th), lambda i, j: (0, j)),
            pl.BlockSpec((1, th), lambda i, j: (0, j)),
            pl.BlockSpec((th, num_experts), lambda i, j: (j, 0)),
            pl.BlockSpec((1, num_experts), lambda i, j: (0, 0)),
        ],
        out_specs=[
            pl.BlockSpec((tm, num_experts), lambda i, j: (i, 0)),
            pl.BlockSpec((tm, num_experts), lambda i, j: (i, 0)),
        ],
        scratch_shapes=[pltpu.VMEM((tm, num_experts), jnp.float32)],
    )

    return pl.pallas_call(
        _router_kernel,
        grid_spec=grid_spec,
        out_shape=out_shapes,
        compiler_params=pltpu.CompilerParams(
            dimension_semantics=("parallel", "arbitrary"),
        ),
    )(flow_input, W1, b1_2d, W2, b2_2d)


def kernel(flow_input, W1, b1, W2, b2):
    return _router(flow_input, W1, b1, W2, b2)


# speedup vs baseline: 1.1854x; 1.0076x over previous
import functools

import jax
import jax.numpy as jnp
from jax.experimental import pallas as pl
from jax.experimental.pallas import tpu as pltpu


def _router_kernel(x_ref, w1_ref, b1_ref, w2_ref, b2_ref,
                   logits_ref, gate_ref, acc_ref):
    j = pl.program_id(1)
    nj = pl.num_programs(1)

    h = jnp.dot(x_ref[...].astype(jnp.bfloat16), w1_ref[...],
                preferred_element_type=jnp.float32)
    h = h + b1_ref[...]
    h = h * jax.nn.sigmoid(h)
    part = jnp.dot(h.astype(jnp.bfloat16), w2_ref[...],
                   preferred_element_type=jnp.float32)

    @pl.when(j == 0)
    def _init():
        acc_ref[...] = part

    @pl.when((j != 0) & (j != nj - 1))
    def _accum():
        acc_ref[...] += part

    @pl.when(j == nj - 1)
    def _finish():
        prev = jnp.where(nj == 1, 0.0, acc_ref[...])
        logits = prev + part + b2_ref[...]
        logits_ref[...] = logits
        m = jnp.max(logits, axis=-1, keepdims=True)
        e = jnp.exp(logits - m)
        gate_ref[...] = e / jnp.sum(e, axis=-1, keepdims=True)


@functools.partial(jax.jit, static_argnames=("tm", "th"))
def _router(flow_input, W1, b1, W2, b2, tm=512, th=2048):
    tokens, d_model = flow_input.shape
    hidden, num_experts = W2.shape
    tm = min(tm, tokens)
    th = min(th, hidden)
    ni = tokens // tm
    nj = hidden // th

    W1 = W1.astype(jnp.bfloat16)
    W2 = W2.astype(jnp.bfloat16)
    b1_2d = b1.reshape(1, hidden)
    b2_2d = b2.reshape(1, num_experts)

    out_shapes = (
        jax.ShapeDtypeStruct((tokens, num_experts), jnp.float32),
        jax.ShapeDtypeStruct((tokens, num_experts), jnp.float32),
    )

    grid_spec = pltpu.PrefetchScalarGridSpec(
        num_scalar_prefetch=0,
        grid=(ni, nj),
        in_specs=[
            pl.BlockSpec((tm, d_model), lambda i, j: (i, 0)),
            pl.BlockSpec((d_model, th), lambda i, j: (0, j)),
            pl.BlockSpec((1, th), lambda i, j: (0, j)),
            pl.BlockSpec((th, num_experts), lambda i, j: (j, 0)),
            pl.BlockSpec((1, num_experts), lambda i, j: (0, 0)),
        ],
        out_specs=[
            pl.BlockSpec((tm, num_experts), lambda i, j: (i, 0)),
            pl.BlockSpec((tm, num_experts), lambda i, j: (i, 0)),
        ],
        scratch_shapes=[pltpu.VMEM((tm, num_experts), jnp.float32)],
    )

    return pl.pallas_call(
        _router_kernel,
        grid_spec=grid_spec,
        out_shape=out_shapes,
        compiler_params=pltpu.CompilerParams(
            dimension_semantics=("parallel", "arbitrary"),
        ),
    )(flow_input, W1, b1_2d, W2, b2_2d)


def kernel(flow_input, W1, b1, W2, b2):
    return _router(flow_input, W1, b1, W2, b2)
